# baseline (speedup 1.0000x reference)
import functools

import jax
import jax.numpy as jnp
from jax import lax
from jax.experimental import pallas as pl
from jax.experimental.pallas import tpu as pltpu
from jax.experimental.pallas import tpu_sc as plsc

N = 10000
E = 320000
D = 128

NC = 2
NS = 16
DH = D // NC
EPS = E // NS
C = 80
NCHUNK = EPS // C
N_PAD = 10240
RPS = N_PAD // NS
F = 128
CW = 16

_f32 = jnp.float32


NB = 5
SEG = 50
NSEG = NCHUNK // SEG


def _sc_body(y_hbm, srcE_hbm, srcO_hbm, dst_hbm, s_out, cnt_out,
             src_v, dst_v, *scratch):
    rows = scratch[:NB]
    ones, zbuf, zcnt, acc_sh, cnt_sh = scratch[NB:NB + 5]
    sems = scratch[NB + 5:]
    cid = lax.axis_index("c")
    sid = lax.axis_index("s")

    def _zrow(i, _):
        for j in range(DH // 16):
            zbuf[i, pl.ds(j * 16, 16)] = jnp.zeros((16,), _f32)
        return 0
    lax.fori_loop(0, F, _zrow, 0)

    def _zcrow(i, _):
        zcnt[i, pl.ds(0, CW)] = jnp.zeros((CW,), _f32)
        return 0
    lax.fori_loop(0, RPS, _zcrow, 0)

    def _orow(i, _):
        ones[i, pl.ds(0, CW)] = jnp.ones((CW,), _f32)
        return 0
    lax.fori_loop(0, C, _orow, 0)

    for k in range(RPS // F):
        pltpu.sync_copy(zbuf, acc_sh.at[pl.ds(sid * RPS + k * F, F)])
    pltpu.sync_copy(zcnt, cnt_sh.at[pl.ds(sid * RPS, RPS)])
    plsc.subcore_barrier()

    for s in range(NSEG):
        count_core = s % 2

        def _scatter(b, j):
            pltpu.make_async_copy(y_hbm.at[src_v.at[b]], rows[b],
                                  sems[b]).wait()
            pltpu.sync_copy(rows[b], acc_sh.at[dst_v.at[j]], add=True)

            @pl.when(cid == count_core)
            def _():
                pltpu.sync_copy(ones, cnt_sh.at[dst_v.at[j]], add=True)

        @pl.when(cid == 0)
        def _():
            pltpu.sync_copy(srcE_hbm.at[sid, pl.ds(s * SEG, SEG)], src_v)

        @pl.when(cid == 1)
        def _():
            pltpu.sync_copy(srcO_hbm.at[sid, pl.ds(s * SEG, SEG)], src_v)

        pltpu.sync_copy(dst_hbm.at[sid, pl.ds(s * SEG, SEG)], dst_v)
        for b in range(NB):
            pltpu.async_copy(y_hbm.at[src_v.at[b]], rows[b], sems[b])

        def _group(g, _):
            j0 = g * NB
            for b in range(NB):
                _scatter(b, j0 + b)
                pltpu.async_copy(y_hbm.at[src_v.at[j0 + b + NB]],
                                 rows[b], sems[b])
            return 0
        lax.fori_loop(0, SEG // NB - 1, _group, 0)
        for b in range(NB):
            _scatter(b, SEG - NB + b)

    plsc.subcore_barrier()

    @pl.when(cid == 0)
    def _():
        for k in range(RPS // F):
            r0 = sid * RPS + k * F
            pltpu.sync_copy(acc_sh.at[pl.ds(r0, F)], s_out.at[pl.ds(r0, F), 0])

    @pl.when(cid == 1)
    def _():
        for k in range(RPS // F):
            r0 = sid * RPS + k * F
            pltpu.sync_copy(acc_sh.at[pl.ds(r0, F)], s_out.at[pl.ds(r0, F), 1])

    pltpu.sync_copy(cnt_sh.at[pl.ds(sid * RPS, RPS)],
                    cnt_out.at[cid, pl.ds(sid * RPS, RPS)])


_sc_agg = functools.partial(
    pl.kernel,
    out_type=(
        jax.ShapeDtypeStruct((N_PAD, NC, DH), _f32),
        jax.ShapeDtypeStruct((NC, N_PAD, CW), _f32),
    ),
    mesh=plsc.VectorSubcoreMesh(core_axis_name="c", subcore_axis_name="s",
                                num_cores=NC, num_subcores=NS),
    scratch_types=[
        pltpu.VMEM((SEG, C), jnp.int32),
        pltpu.VMEM((SEG, C), jnp.int32),
    ] + [pltpu.VMEM((C, DH), _f32) for _ in range(NB)] + [
        pltpu.VMEM((C, CW), _f32),
        pltpu.VMEM((F, DH), _f32),
        pltpu.VMEM((RPS, CW), _f32),
        pltpu.VMEM_SHARED((N_PAD, DH), _f32),
        pltpu.VMEM_SHARED((N_PAD, CW), _f32),
    ] + [pltpu.SemaphoreType.DMA for _ in range(NB)],
    compiler_params=pltpu.CompilerParams(use_tc_tiling_on_sc=False),
)(_sc_body)


_RB = 1024
_GRID = (N + _RB - 1) // _RB


def _dense0_body(x_ref, wl_ref, wr_ref, bl_ref, br_ref, y_ref, r_ref):
    xb = x_ref[...]
    y_ref[...] = jnp.dot(xb, wl_ref[...], preferred_element_type=_f32)
    r_ref[...] = (jnp.dot(xb, wr_ref[...], preferred_element_type=_f32)
                  + bl_ref[...] + br_ref[...])


_dense0 = pl.pallas_call(
    _dense0_body,
    grid=(_GRID,),
    in_specs=[
        pl.BlockSpec((_RB, D), lambda i: (i, 0)),
        pl.BlockSpec((D, D), lambda i: (0, 0)),
        pl.BlockSpec((D, D), lambda i: (0, 0)),
        pl.BlockSpec((1, D), lambda i: (0, 0)),
        pl.BlockSpec((1, D), lambda i: (0, 0)),
    ],
    out_specs=[
        pl.BlockSpec((_RB, D), lambda i: (i, 0)),
        pl.BlockSpec((_RB, D), lambda i: (i, 0)),
    ],
    out_shape=[
        jax.ShapeDtypeStruct((N, D), _f32),
        jax.ShapeDtypeStruct((N, D), _f32),
    ],
)


def _mean(s_ref, c_ref):
    cnt = c_ref[0, :, 0:1] + c_ref[1, :, 0:1]
    return s_ref[...] / jnp.maximum(cnt, 1.0)


def _combine_mid_body(s_ref, c_ref, r_ref, wl_ref, wr_ref,
                      bl_ref, br_ref, y_ref, rn_ref):
    h = jnp.maximum(_mean(s_ref, c_ref) + r_ref[...], 0.0)
    y_ref[...] = jnp.dot(h, wl_ref[...], preferred_element_type=_f32)
    rn_ref[...] = (jnp.dot(h, wr_ref[...], preferred_element_type=_f32)
                   + bl_ref[...] + br_ref[...])


_combine_mid = pl.pallas_call(
    _combine_mid_body,
    grid=(_GRID,),
    in_specs=[
        pl.BlockSpec((_RB, D), lambda i: (i, 0)),
        pl.BlockSpec((NC, _RB, CW), lambda i: (0, i, 0)),
        pl.BlockSpec((_RB, D), lambda i: (i, 0)),
        pl.BlockSpec((D, D), lambda i: (0, 0)),
        pl.BlockSpec((D, D), lambda i: (0, 0)),
        pl.BlockSpec((1, D), lambda i: (0, 0)),
        pl.BlockSpec((1, D), lambda i: (0, 0)),
    ],
    out_specs=[
        pl.BlockSpec((_RB, D), lambda i: (i, 0)),
        pl.BlockSpec((_RB, D), lambda i: (i, 0)),
    ],
    out_shape=[
        jax.ShapeDtypeStruct((N, D), _f32),
        jax.ShapeDtypeStruct((N, D), _f32),
    ],
)


def _combine_out_body(s_ref, c_ref, r_ref, o_ref):
    z = _mean(s_ref, c_ref) + r_ref[...]
    m = jnp.max(z, axis=-1, keepdims=True)
    e = jnp.exp(z - m)
    o_ref[...] = (z - m) - jnp.log(jnp.sum(e, axis=-1, keepdims=True))


_combine_out = pl.pallas_call(
    _combine_out_body,
    grid=(_GRID,),
    in_specs=[
        pl.BlockSpec((_RB, D), lambda i: (i, 0)),
        pl.BlockSpec((NC, _RB, CW), lambda i: (0, i, 0)),
        pl.BlockSpec((_RB, D), lambda i: (i, 0)),
    ],
    out_specs=pl.BlockSpec((_RB, D), lambda i: (i, 0)),
    out_shape=jax.ShapeDtypeStruct((N, D), _f32),
)


def kernel(x, edge_index_l0, edge_index_l1,
           W_l0, b_l0, W_r0, b_r0,
           W_l1, b_l1, W_r1, b_r1):
    def _idx(edge_index):
        src = edge_index[0]
        return ((2 * src).reshape(NS, NCHUNK, C),
                (2 * src + 1).reshape(NS, NCHUNK, C),
                edge_index[1].reshape(NS, NCHUNK, C))

    srcE0, srcO0, dst0 = _idx(edge_index_l0)
    srcE1, srcO1, dst1 = _idx(edge_index_l1)
    bl0 = b_l0.reshape(1, D)
    br0 = b_r0.reshape(1, D)
    bl1 = b_l1.reshape(1, D)
    br1 = b_r1.reshape(1, D)

    y0, r0 = _dense0(x, W_l0, W_r0, bl0, br0)
    s0, c0 = _sc_agg(y0.reshape(2 * N, DH), srcE0, srcO0, dst0)
    y1, r1 = _combine_mid(s0.reshape(N_PAD, D), c0, r0,
                          W_l1, W_r1, bl1, br1)
    s1, c1 = _sc_agg(y1.reshape(2 * N, DH), srcE1, srcO1, dst1)
    return _combine_out(s1.reshape(N_PAD, D), c1, r1)


# speedup vs baseline: 1.2902x; 1.2902x over previous
import functools

import jax
import jax.numpy as jnp
from jax import lax
from jax.experimental import pallas as pl
from jax.experimental.pallas import tpu as pltpu
from jax.experimental.pallas import tpu_sc as plsc

N = 10000
E = 320000
D = 128

NC = 2
NS = 16
DH = D // NC
EPS = E // NS
C = 80
NCHUNK = EPS // C
N_PAD = 10240
RPS = N_PAD // NS
F = 128
CW = 16

_f32 = jnp.float32


NB = 5
SEG = 50
NSEG = NCHUNK // SEG


def _sc_body(y_hbm, srcE_hbm, srcO_hbm, dst_hbm, s_out, cnt_out,
             src_v, dst_v, *scratch):
    rows = scratch[:NB]
    ones, zbuf, zcnt, acc_sh, cnt_sh = scratch[NB:NB + 5]
    sems = scratch[NB + 5:]
    cid = lax.axis_index("c")
    sid = lax.axis_index("s")

    def _zrow(i, _):
        for j in range(DH // 16):
            zbuf[i, pl.ds(j * 16, 16)] = jnp.zeros((16,), _f32)
        return 0
    lax.fori_loop(0, F, _zrow, 0)

    def _zcrow(i, _):
        zcnt[i, pl.ds(0, CW)] = jnp.zeros((CW,), _f32)
        return 0
    lax.fori_loop(0, RPS, _zcrow, 0)

    def _orow(i, _):
        ones[i, pl.ds(0, CW)] = jnp.ones((CW,), _f32)
        return 0
    lax.fori_loop(0, C, _orow, 0)

    for k in range(RPS // F):
        pltpu.sync_copy(zbuf, acc_sh.at[pl.ds(sid * RPS + k * F, F)])
    pltpu.sync_copy(zcnt, cnt_sh.at[pl.ds(sid * RPS, RPS)])
    plsc.subcore_barrier()

    for s in range(NSEG):
        count_core = s % 2

        def _scatter(b, j):
            pltpu.make_async_copy(y_hbm.at[src_v.at[b]], rows[b],
                                  sems[b]).wait()
            pltpu.sync_copy(rows[b], acc_sh.at[dst_v.at[j]], add=True)

            @pl.when(cid == count_core)
            def _():
                pltpu.sync_copy(ones, cnt_sh.at[dst_v.at[j]], add=True)

        @pl.when(cid == 0)
        def _():
            pltpu.sync_copy(srcE_hbm.at[sid, pl.ds(s * SEG, SEG)], src_v)

        @pl.when(cid == 1)
        def _():
            pltpu.sync_copy(srcO_hbm.at[sid, pl.ds(s * SEG, SEG)], src_v)

        pltpu.sync_copy(dst_hbm.at[sid, pl.ds(s * SEG, SEG)], dst_v)
        for b in range(NB):
            pltpu.async_copy(y_hbm.at[src_v.at[b]], rows[b], sems[b])

        def _group(g, _):
            j0 = g * NB
            for b in range(NB):
                _scatter(b, j0 + b)
                pltpu.async_copy(y_hbm.at[src_v.at[j0 + b + NB]],
                                 rows[b], sems[b])
            return 0
        lax.fori_loop(0, SEG // NB - 1, _group, 0)
        for b in range(NB):
            _scatter(b, SEG - NB + b)

    plsc.subcore_barrier()

    @pl.when(cid == 0)
    def _():
        for k in range(RPS // F):
            r0 = sid * RPS + k * F
            pltpu.sync_copy(acc_sh.at[pl.ds(r0, F)],
                            s_out.at[pl.ds(r0, F), pl.ds(0, DH)])
        pltpu.sync_copy(cnt_sh.at[pl.ds(sid * RPS, RPS)],
                        cnt_out.at[pl.ds(sid * RPS, RPS), pl.ds(0, CW)])

    @pl.when(cid == 1)
    def _():
        for k in range(RPS // F):
            r0 = sid * RPS + k * F
            pltpu.sync_copy(acc_sh.at[pl.ds(r0, F)],
                            s_out.at[pl.ds(r0, F), pl.ds(DH, DH)])
        pltpu.sync_copy(cnt_sh.at[pl.ds(sid * RPS, RPS)],
                        cnt_out.at[pl.ds(sid * RPS, RPS), pl.ds(CW, CW)])


_sc_agg = functools.partial(
    pl.kernel,
    out_type=(
        jax.ShapeDtypeStruct((N_PAD, D), _f32),
        jax.ShapeDtypeStruct((N_PAD, D), _f32),
    ),
    mesh=plsc.VectorSubcoreMesh(core_axis_name="c", subcore_axis_name="s",
                                num_cores=NC, num_subcores=NS),
    scratch_types=[
        pltpu.VMEM((SEG, C), jnp.int32),
        pltpu.VMEM((SEG, C), jnp.int32),
    ] + [pltpu.VMEM((C, DH), _f32) for _ in range(NB)] + [
        pltpu.VMEM((C, CW), _f32),
        pltpu.VMEM((F, DH), _f32),
        pltpu.VMEM((RPS, CW), _f32),
        pltpu.VMEM_SHARED((N_PAD, DH), _f32),
        pltpu.VMEM_SHARED((N_PAD, CW), _f32),
    ] + [pltpu.SemaphoreType.DMA for _ in range(NB)],
    compiler_params=pltpu.CompilerParams(use_tc_tiling_on_sc=False),
)(_sc_body)


_RB = 1024
_GRID = (N + _RB - 1) // _RB


def _dense0_body(x_ref, wl_ref, wr_ref, bl_ref, br_ref, y_ref, r_ref):
    xb = x_ref[...]
    y_ref[...] = jnp.dot(xb, wl_ref[...], preferred_element_type=_f32)
    r_ref[...] = (jnp.dot(xb, wr_ref[...], preferred_element_type=_f32)
                  + bl_ref[...] + br_ref[...])


_dense0 = pl.pallas_call(
    _dense0_body,
    grid=(_GRID,),
    in_specs=[
        pl.BlockSpec((_RB, D), lambda i: (i, 0)),
        pl.BlockSpec((D, D), lambda i: (0, 0)),
        pl.BlockSpec((D, D), lambda i: (0, 0)),
        pl.BlockSpec((1, D), lambda i: (0, 0)),
        pl.BlockSpec((1, D), lambda i: (0, 0)),
    ],
    out_specs=[
        pl.BlockSpec((_RB, D), lambda i: (i, 0)),
        pl.BlockSpec((_RB, D), lambda i: (i, 0)),
    ],
    out_shape=[
        jax.ShapeDtypeStruct((N, D), _f32),
        jax.ShapeDtypeStruct((N, D), _f32),
    ],
)


def _mean(s_ref, c_ref):
    cnt = c_ref[:, 0:1] + c_ref[:, CW:CW + 1]
    return s_ref[...] / jnp.maximum(cnt, 1.0)


def _combine_mid_body(s_ref, c_ref, r_ref, wl_ref, wr_ref,
                      bl_ref, br_ref, y_ref, rn_ref):
    h = jnp.maximum(_mean(s_ref, c_ref) + r_ref[...], 0.0)
    y_ref[...] = jnp.dot(h, wl_ref[...], preferred_element_type=_f32)
    rn_ref[...] = (jnp.dot(h, wr_ref[...], preferred_element_type=_f32)
                   + bl_ref[...] + br_ref[...])


_combine_mid = pl.pallas_call(
    _combine_mid_body,
    grid=(_GRID,),
    in_specs=[
        pl.BlockSpec((_RB, D), lambda i: (i, 0)),
        pl.BlockSpec((_RB, D), lambda i: (i, 0)),
        pl.BlockSpec((_RB, D), lambda i: (i, 0)),
        pl.BlockSpec((D, D), lambda i: (0, 0)),
        pl.BlockSpec((D, D), lambda i: (0, 0)),
        pl.BlockSpec((1, D), lambda i: (0, 0)),
        pl.BlockSpec((1, D), lambda i: (0, 0)),
    ],
    out_specs=[
        pl.BlockSpec((_RB, D), lambda i: (i, 0)),
        pl.BlockSpec((_RB, D), lambda i: (i, 0)),
    ],
    out_shape=[
        jax.ShapeDtypeStruct((N, D), _f32),
        jax.ShapeDtypeStruct((N, D), _f32),
    ],
)


def _combine_out_body(s_ref, c_ref, r_ref, o_ref):
    z = _mean(s_ref, c_ref) + r_ref[...]
    m = jnp.max(z, axis=-1, keepdims=True)
    e = jnp.exp(z - m)
    o_ref[...] = (z - m) - jnp.log(jnp.sum(e, axis=-1, keepdims=True))


_combine_out = pl.pallas_call(
    _combine_out_body,
    grid=(_GRID,),
    in_specs=[
        pl.BlockSpec((_RB, D), lambda i: (i, 0)),
        pl.BlockSpec((_RB, D), lambda i: (i, 0)),
        pl.BlockSpec((_RB, D), lambda i: (i, 0)),
    ],
    out_specs=pl.BlockSpec((_RB, D), lambda i: (i, 0)),
    out_shape=jax.ShapeDtypeStruct((N, D), _f32),
)


def kernel(x, edge_index_l0, edge_index_l1,
           W_l0, b_l0, W_r0, b_r0,
           W_l1, b_l1, W_r1, b_r1):
    def _idx(edge_index):
        src = edge_index[0]
        return ((2 * src).reshape(NS, NCHUNK, C),
                (2 * src + 1).reshape(NS, NCHUNK, C),
                edge_index[1].reshape(NS, NCHUNK, C))

    srcE0, srcO0, dst0 = _idx(edge_index_l0)
    srcE1, srcO1, dst1 = _idx(edge_index_l1)
    bl0 = b_l0.reshape(1, D)
    br0 = b_r0.reshape(1, D)
    bl1 = b_l1.reshape(1, D)
    br1 = b_r1.reshape(1, D)

    y0, r0 = _dense0(x, W_l0, W_r0, bl0, br0)
    s0, c0 = _sc_agg(y0.reshape(2 * N, DH), srcE0, srcO0, dst0)
    y1, r1 = _combine_mid(s0, c0, r0, W_l1, W_r1, bl1, br1)
    s1, c1 = _sc_agg(y1.reshape(2 * N, DH), srcE1, srcO1, dst1)
    return _combine_out(s1, c1, r1)


# speedup vs baseline: 1.3763x; 1.0668x over previous
import functools

import jax
import jax.numpy as jnp
from jax import lax
from jax.experimental import pallas as pl
from jax.experimental.pallas import tpu as pltpu
from jax.experimental.pallas import tpu_sc as plsc

N = 10000
E = 320000
D = 128

NC = 2
NS = 16
DH = D // NC
EPS = E // NS
C = 80
NCHUNK = EPS // C
N_PAD = 10240
RPS = N_PAD // NS
F = 128
CW = 16

_f32 = jnp.float32


NB = 5
SEG = 125
NSEG = NCHUNK // SEG


def _sc_body(y_hbm, srcE_hbm, srcO_hbm, dst_hbm, s_out, cnt_out,
             src_v, dst_v, *scratch):
    rows = scratch[:NB]
    ones, zbuf, zcnt, acc_sh, cnt_sh = scratch[NB:NB + 5]
    sems = scratch[NB + 5:]
    cid = lax.axis_index("c")
    sid = lax.axis_index("s")

    def _zrow(i, _):
        for j in range(DH // 16):
            zbuf[i, pl.ds(j * 16, 16)] = jnp.zeros((16,), _f32)
        return 0
    lax.fori_loop(0, F, _zrow, 0)

    def _zcrow(i, _):
        zcnt[i, pl.ds(0, CW)] = jnp.zeros((CW,), _f32)
        return 0
    lax.fori_loop(0, RPS, _zcrow, 0)

    def _orow(i, _):
        ones[i, pl.ds(0, CW)] = jnp.ones((CW,), _f32)
        return 0
    lax.fori_loop(0, C, _orow, 0)

    for k in range(RPS // F):
        pltpu.sync_copy(zbuf, acc_sh.at[pl.ds(sid * RPS + k * F, F)])
    pltpu.sync_copy(zcnt, cnt_sh.at[pl.ds(sid * RPS, RPS)])
    plsc.subcore_barrier()

    for s in range(NSEG):
        count_core = s % 2

        def _scatter(b, j):
            pltpu.make_async_copy(y_hbm.at[src_v.at[b]], rows[b],
                                  sems[b]).wait()
            pltpu.sync_copy(rows[b], acc_sh.at[dst_v.at[j]], add=True)

            @pl.when(cid == count_core)
            def _():
                pltpu.sync_copy(ones, cnt_sh.at[dst_v.at[j]], add=True)

        @pl.when(cid == 0)
        def _():
            pltpu.sync_copy(srcE_hbm.at[sid, pl.ds(s * SEG, SEG)], src_v)

        @pl.when(cid == 1)
        def _():
            pltpu.sync_copy(srcO_hbm.at[sid, pl.ds(s * SEG, SEG)], src_v)

        pltpu.sync_copy(dst_hbm.at[sid, pl.ds(s * SEG, SEG)], dst_v)
        for b in range(NB):
            pltpu.async_copy(y_hbm.at[src_v.at[b]], rows[b], sems[b])

        def _group(g, _):
            j0 = g * NB
            for b in range(NB):
                _scatter(b, j0 + b)
                pltpu.async_copy(y_hbm.at[src_v.at[j0 + b + NB]],
                                 rows[b], sems[b])
            return 0
        lax.fori_loop(0, SEG // NB - 1, _group, 0)
        for b in range(NB):
            _scatter(b, SEG - NB + b)

    plsc.subcore_barrier()

    @pl.when(cid == 0)
    def _():
        for k in range(RPS // F):
            r0 = sid * RPS + k * F
            pltpu.sync_copy(acc_sh.at[pl.ds(r0, F)],
                            s_out.at[pl.ds(r0, F), pl.ds(0, DH)])
        pltpu.sync_copy(cnt_sh.at[pl.ds(sid * RPS, RPS)],
                        cnt_out.at[pl.ds(sid * RPS, RPS), pl.ds(0, CW)])

    @pl.when(cid == 1)
    def _():
        for k in range(RPS // F):
            r0 = sid * RPS + k * F
            pltpu.sync_copy(acc_sh.at[pl.ds(r0, F)],
                            s_out.at[pl.ds(r0, F), pl.ds(DH, DH)])
        pltpu.sync_copy(cnt_sh.at[pl.ds(sid * RPS, RPS)],
                        cnt_out.at[pl.ds(sid * RPS, RPS), pl.ds(CW, CW)])


_sc_agg = functools.partial(
    pl.kernel,
    out_type=(
        jax.ShapeDtypeStruct((N_PAD, D), _f32),
        jax.ShapeDtypeStruct((N_PAD, D), _f32),
    ),
    mesh=plsc.VectorSubcoreMesh(core_axis_name="c", subcore_axis_name="s",
                                num_cores=NC, num_subcores=NS),
    scratch_types=[
        pltpu.VMEM((SEG, C), jnp.int32),
        pltpu.VMEM((SEG, C), jnp.int32),
    ] + [pltpu.VMEM((C, DH), _f32) for _ in range(NB)] + [
        pltpu.VMEM((C, CW), _f32),
        pltpu.VMEM((F, DH), _f32),
        pltpu.VMEM((RPS, CW), _f32),
        pltpu.VMEM_SHARED((N_PAD, DH), _f32),
        pltpu.VMEM_SHARED((N_PAD, CW), _f32),
    ] + [pltpu.SemaphoreType.DMA for _ in range(NB)],
    compiler_params=pltpu.CompilerParams(use_tc_tiling_on_sc=False),
)(_sc_body)


_RB = 1024
_GRID = (N + _RB - 1) // _RB


def _dense0_body(x_ref, wl_ref, wr_ref, bl_ref, br_ref, y_ref, r_ref):
    xb = x_ref[...]
    y_ref[...] = jnp.dot(xb, wl_ref[...], preferred_element_type=_f32)
    r_ref[...] = (jnp.dot(xb, wr_ref[...], preferred_element_type=_f32)
                  + bl_ref[...] + br_ref[...])


_dense0 = pl.pallas_call(
    _dense0_body,
    grid=(_GRID,),
    in_specs=[
        pl.BlockSpec((_RB, D), lambda i: (i, 0)),
        pl.BlockSpec((D, D), lambda i: (0, 0)),
        pl.BlockSpec((D, D), lambda i: (0, 0)),
        pl.BlockSpec((1, D), lambda i: (0, 0)),
        pl.BlockSpec((1, D), lambda i: (0, 0)),
    ],
    out_specs=[
        pl.BlockSpec((_RB, D), lambda i: (i, 0)),
        pl.BlockSpec((_RB, D), lambda i: (i, 0)),
    ],
    out_shape=[
        jax.ShapeDtypeStruct((N, D), _f32),
        jax.ShapeDtypeStruct((N, D), _f32),
    ],
)


def _mean(s_ref, c_ref):
    cnt = c_ref[:, 0:1] + c_ref[:, CW:CW + 1]
    return s_ref[...] / jnp.maximum(cnt, 1.0)


def _combine_mid_body(s_ref, c_ref, r_ref, wl_ref, wr_ref,
                      bl_ref, br_ref, y_ref, rn_ref):
    h = jnp.maximum(_mean(s_ref, c_ref) + r_ref[...], 0.0)
    y_ref[...] = jnp.dot(h, wl_ref[...], preferred_element_type=_f32)
    rn_ref[...] = (jnp.dot(h, wr_ref[...], preferred_element_type=_f32)
                   + bl_ref[...] + br_ref[...])


_combine_mid = pl.pallas_call(
    _combine_mid_body,
    grid=(_GRID,),
    in_specs=[
        pl.BlockSpec((_RB, D), lambda i: (i, 0)),
        pl.BlockSpec((_RB, D), lambda i: (i, 0)),
        pl.BlockSpec((_RB, D), lambda i: (i, 0)),
        pl.BlockSpec((D, D), lambda i: (0, 0)),
        pl.BlockSpec((D, D), lambda i: (0, 0)),
        pl.BlockSpec((1, D), lambda i: (0, 0)),
        pl.BlockSpec((1, D), lambda i: (0, 0)),
    ],
    out_specs=[
        pl.BlockSpec((_RB, D), lambda i: (i, 0)),
        pl.BlockSpec((_RB, D), lambda i: (i, 0)),
    ],
    out_shape=[
        jax.ShapeDtypeStruct((N, D), _f32),
        jax.ShapeDtypeStruct((N, D), _f32),
    ],
)


def _combine_out_body(s_ref, c_ref, r_ref, o_ref):
    z = _mean(s_ref, c_ref) + r_ref[...]
    m = jnp.max(z, axis=-1, keepdims=True)
    e = jnp.exp(z - m)
    o_ref[...] = (z - m) - jnp.log(jnp.sum(e, axis=-1, keepdims=True))


_combine_out = pl.pallas_call(
    _combine_out_body,
    grid=(_GRID,),
    in_specs=[
        pl.BlockSpec((_RB, D), lambda i: (i, 0)),
        pl.BlockSpec((_RB, D), lambda i: (i, 0)),
        pl.BlockSpec((_RB, D), lambda i: (i, 0)),
    ],
    out_specs=pl.BlockSpec((_RB, D), lambda i: (i, 0)),
    out_shape=jax.ShapeDtypeStruct((N, D), _f32),
)


def kernel(x, edge_index_l0, edge_index_l1,
           W_l0, b_l0, W_r0, b_r0,
           W_l1, b_l1, W_r1, b_r1):
    def _idx(edge_index):
        src = edge_index[0]
        return ((2 * src).reshape(NS, NCHUNK, C),
                (2 * src + 1).reshape(NS, NCHUNK, C),
                edge_index[1].reshape(NS, NCHUNK, C))

    srcE0, srcO0, dst0 = _idx(edge_index_l0)
    srcE1, srcO1, dst1 = _idx(edge_index_l1)
    bl0 = b_l0.reshape(1, D)
    br0 = b_r0.reshape(1, D)
    bl1 = b_l1.reshape(1, D)
    br1 = b_r1.reshape(1, D)

    y0, r0 = _dense0(x, W_l0, W_r0, bl0, br0)
    s0, c0 = _sc_agg(y0.reshape(2 * N, DH), srcE0, srcO0, dst0)
    y1, r1 = _combine_mid(s0, c0, r0, W_l1, W_r1, bl1, br1)
    s1, c1 = _sc_agg(y1.reshape(2 * N, DH), srcE1, srcO1, dst1)
    return _combine_out(s1, c1, r1)


# speedup vs baseline: 1.4456x; 1.0503x over previous
import functools

import jax
import jax.numpy as jnp
from jax import lax
from jax.experimental import pallas as pl
from jax.experimental.pallas import tpu as pltpu
from jax.experimental.pallas import tpu_sc as plsc

N = 10000
E = 320000
D = 128

NC = 2
NS = 16
DH = D // NC
C = 128
NCHT = E // C
CPS = NCHT // NS
XTRA = NCHT - CPS * NS
N_PAD = 10240
RPS = N_PAD // NS
F = 128
CW = 16

_f32 = jnp.float32


NB = 4
SEG = 52
NSEG = CPS // SEG


def _sc_body(y_hbm, ei_hbm, s_out, cnt_out, idx_v, *scratch):
    rows = scratch[:NB]
    ones, zbuf, zcnt, acc_sh, cnt_sh = scratch[NB:NB + 5]
    sems = scratch[NB + 5:]
    cid = lax.axis_index("c")
    sid = lax.axis_index("s")

    def _zrow(i, _):
        for j in range(DH // 16):
            zbuf[i, pl.ds(j * 16, 16)] = jnp.zeros((16,), _f32)
        return 0
    lax.fori_loop(0, F, _zrow, 0)

    def _zcrow(i, _):
        zcnt[i, pl.ds(0, CW)] = jnp.zeros((CW,), _f32)
        return 0
    lax.fori_loop(0, RPS, _zcrow, 0)

    def _orow(i, _):
        ones[i, pl.ds(0, CW)] = jnp.ones((CW,), _f32)
        return 0
    lax.fori_loop(0, C, _orow, 0)

    for k in range(RPS // F):
        pltpu.sync_copy(zbuf, acc_sh.at[pl.ds(sid * RPS + k * F, F)])
    pltpu.sync_copy(zcnt, cnt_sh.at[pl.ds(sid * RPS, RPS)])
    plsc.subcore_barrier()

    start = sid * CPS + jnp.minimum(sid, XTRA)

    def _xform(j):
        for k in range(C // 16):
            v = idx_v[j, 0, pl.ds(k * 16, 16)]
            idx_v[j, 0, pl.ds(k * 16, 16)] = v + v + cid

    def _gissue(b, j):
        _xform(j)
        pltpu.async_copy(y_hbm.at[idx_v.at[j, 0]], rows[b], sems[b])

    def _scatter(b, j, count_core):
        pltpu.make_async_copy(y_hbm.at[idx_v.at[0, 0]], rows[b],
                              sems[b]).wait()
        pltpu.sync_copy(rows[b], acc_sh.at[idx_v.at[j, 1]], add=True)

        @pl.when(cid == count_core)
        def _():
            pltpu.sync_copy(ones, cnt_sh.at[idx_v.at[j, 1]], add=True)

    for s in range(NSEG):
        pltpu.sync_copy(ei_hbm.at[pl.ds(start + s * SEG, SEG)], idx_v)
        for b in range(NB):
            _gissue(b, b)

        def _group(g, _):
            j0 = g * NB
            for b in range(NB):
                _scatter(b, j0 + b, g % 2)
                _gissue(b, j0 + b + NB)
            return 0
        lax.fori_loop(0, SEG // NB - 1, _group, 0)
        for b in range(NB):
            _scatter(b, SEG - NB + b, (SEG // NB - 1) % 2)

    @pl.when(sid < XTRA)
    def _():
        pltpu.sync_copy(ei_hbm.at[pl.ds(start + CPS, 1)], idx_v.at[pl.ds(0, 1)])
        _xform(0)
        pltpu.async_copy(y_hbm.at[idx_v.at[0, 0]], rows[0], sems[0]).wait()
        pltpu.sync_copy(rows[0], acc_sh.at[idx_v.at[0, 1]], add=True)

        @pl.when(cid == 0)
        def _():
            pltpu.sync_copy(ones, cnt_sh.at[idx_v.at[0, 1]], add=True)

    plsc.subcore_barrier()

    @pl.when(cid == 0)
    def _():
        for k in range(RPS // F):
            r0 = sid * RPS + k * F
            pltpu.sync_copy(acc_sh.at[pl.ds(r0, F)],
                            s_out.at[pl.ds(r0, F), pl.ds(0, DH)])
        pltpu.sync_copy(cnt_sh.at[pl.ds(sid * RPS, RPS)],
                        cnt_out.at[pl.ds(sid * RPS, RPS), pl.ds(0, CW)])

    @pl.when(cid == 1)
    def _():
        for k in range(RPS // F):
            r0 = sid * RPS + k * F
            pltpu.sync_copy(acc_sh.at[pl.ds(r0, F)],
                            s_out.at[pl.ds(r0, F), pl.ds(DH, DH)])
        pltpu.sync_copy(cnt_sh.at[pl.ds(sid * RPS, RPS)],
                        cnt_out.at[pl.ds(sid * RPS, RPS), pl.ds(CW, CW)])


_sc_agg = functools.partial(
    pl.kernel,
    out_type=(
        jax.ShapeDtypeStruct((N_PAD, D), _f32),
        jax.ShapeDtypeStruct((N_PAD, D), _f32),
    ),
    mesh=plsc.VectorSubcoreMesh(core_axis_name="c", subcore_axis_name="s",
                                num_cores=NC, num_subcores=NS),
    scratch_types=[
        pltpu.VMEM((SEG, 2, C), jnp.int32),
    ] + [pltpu.VMEM((C, DH), _f32) for _ in range(NB)] + [
        pltpu.VMEM((C, CW), _f32),
        pltpu.VMEM((F, DH), _f32),
        pltpu.VMEM((RPS, CW), _f32),
        pltpu.VMEM_SHARED((N_PAD, DH), _f32),
        pltpu.VMEM_SHARED((N_PAD, CW), _f32),
    ] + [pltpu.SemaphoreType.DMA for _ in range(NB)],
    compiler_params=pltpu.CompilerParams(use_tc_tiling_on_sc=False),
)(_sc_body)


_RB = 1024
_GRID = (N + _RB - 1) // _RB


def _dense0_body(x_ref, wl_ref, wr_ref, bl_ref, br_ref, y_ref, r_ref):
    xb = x_ref[...]
    y_ref[...] = jnp.dot(xb, wl_ref[...], preferred_element_type=_f32)
    r_ref[...] = (jnp.dot(xb, wr_ref[...], preferred_element_type=_f32)
                  + bl_ref[...] + br_ref[...])


_dense0 = pl.pallas_call(
    _dense0_body,
    grid=(_GRID,),
    in_specs=[
        pl.BlockSpec((_RB, D), lambda i: (i, 0)),
        pl.BlockSpec((D, D), lambda i: (0, 0)),
        pl.BlockSpec((D, D), lambda i: (0, 0)),
        pl.BlockSpec((1, D), lambda i: (0, 0)),
        pl.BlockSpec((1, D), lambda i: (0, 0)),
    ],
    out_specs=[
        pl.BlockSpec((_RB, D), lambda i: (i, 0)),
        pl.BlockSpec((_RB, D), lambda i: (i, 0)),
    ],
    out_shape=[
        jax.ShapeDtypeStruct((N, D), _f32),
        jax.ShapeDtypeStruct((N, D), _f32),
    ],
)


def _mean(s_ref, c_ref):
    cnt = c_ref[:, 0:1] + c_ref[:, CW:CW + 1]
    return s_ref[...] / jnp.maximum(cnt, 1.0)


def _combine_mid_body(s_ref, c_ref, r_ref, wl_ref, wr_ref,
                      bl_ref, br_ref, y_ref, rn_ref):
    h = jnp.maximum(_mean(s_ref, c_ref) + r_ref[...], 0.0)
    y_ref[...] = jnp.dot(h, wl_ref[...], preferred_element_type=_f32)
    rn_ref[...] = (jnp.dot(h, wr_ref[...], preferred_element_type=_f32)
                   + bl_ref[...] + br_ref[...])


_combine_mid = pl.pallas_call(
    _combine_mid_body,
    grid=(_GRID,),
    in_specs=[
        pl.BlockSpec((_RB, D), lambda i: (i, 0)),
        pl.BlockSpec((_RB, D), lambda i: (i, 0)),
        pl.BlockSpec((_RB, D), lambda i: (i, 0)),
        pl.BlockSpec((D, D), lambda i: (0, 0)),
        pl.BlockSpec((D, D), lambda i: (0, 0)),
        pl.BlockSpec((1, D), lambda i: (0, 0)),
        pl.BlockSpec((1, D), lambda i: (0, 0)),
    ],
    out_specs=[
        pl.BlockSpec((_RB, D), lambda i: (i, 0)),
        pl.BlockSpec((_RB, D), lambda i: (i, 0)),
    ],
    out_shape=[
        jax.ShapeDtypeStruct((N, D), _f32),
        jax.ShapeDtypeStruct((N, D), _f32),
    ],
)


def _combine_out_body(s_ref, c_ref, r_ref, o_ref):
    z = _mean(s_ref, c_ref) + r_ref[...]
    m = jnp.max(z, axis=-1, keepdims=True)
    e = jnp.exp(z - m)
    o_ref[...] = (z - m) - jnp.log(jnp.sum(e, axis=-1, keepdims=True))


_combine_out = pl.pallas_call(
    _combine_out_body,
    grid=(_GRID,),
    in_specs=[
        pl.BlockSpec((_RB, D), lambda i: (i, 0)),
        pl.BlockSpec((_RB, D), lambda i: (i, 0)),
        pl.BlockSpec((_RB, D), lambda i: (i, 0)),
    ],
    out_specs=pl.BlockSpec((_RB, D), lambda i: (i, 0)),
    out_shape=jax.ShapeDtypeStruct((N, D), _f32),
)


def kernel(x, edge_index_l0, edge_index_l1,
           W_l0, b_l0, W_r0, b_r0,
           W_l1, b_l1, W_r1, b_r1):
    ei0 = edge_index_l0.reshape(2, NCHT, C).transpose(1, 0, 2)
    ei1 = edge_index_l1.reshape(2, NCHT, C).transpose(1, 0, 2)
    bl0 = b_l0.reshape(1, D)
    br0 = b_r0.reshape(1, D)
    bl1 = b_l1.reshape(1, D)
    br1 = b_r1.reshape(1, D)

    y0, r0 = _dense0(x, W_l0, W_r0, bl0, br0)
    s0, c0 = _sc_agg(y0.reshape(2 * N, DH), ei0)
    y1, r1 = _combine_mid(s0, c0, r0, W_l1, W_r1, bl1, br1)
    s1, c1 = _sc_agg(y1.reshape(2 * N, DH), ei1)
    return _combine_out(s1, c1, r1)


# speedup vs baseline: 1.4666x; 1.0146x over previous
import functools

import jax
import jax.numpy as jnp
from jax import lax
from jax.experimental import pallas as pl
from jax.experimental.pallas import tpu as pltpu
from jax.experimental.pallas import tpu_sc as plsc

N = 10000
E = 320000
D = 128

NC = 2
NS = 16
DH = D // NC
C = 128
NCHT = E // C
CPS = NCHT // NS
XTRA = NCHT - CPS * NS
N_PAD = 10240
RPS = N_PAD // NS
F = 128
CW = 16

_f32 = jnp.float32


NB = 4
SEG = 52
NSEG = CPS // SEG


def _sc_body(y_hbm, ei_hbm, s_out, cnt_out, idx_v, *scratch):
    rows = scratch[:NB]
    ones, zbuf, zcnt, acc_sh, cnt_sh = scratch[NB:NB + 5]
    sems = scratch[NB + 5:]
    cid = lax.axis_index("c")
    sid = lax.axis_index("s")
    start = sid * CPS + jnp.minimum(sid, XTRA)

    def _xform(j):
        for k in range(C // 16):
            v = idx_v[j, 0, pl.ds(k * 16, 16)]
            idx_v[j, 0, pl.ds(k * 16, 16)] = v + v + cid

    def _gissue(b, j):
        _xform(j)
        pltpu.async_copy(y_hbm.at[idx_v.at[j, 0]], rows[b], sems[b])

    pltpu.sync_copy(ei_hbm.at[pl.ds(start, SEG)], idx_v)
    for b in range(NB):
        _gissue(b, b)

    def _zrow(i, _):
        for j in range(DH // 16):
            zbuf[i, pl.ds(j * 16, 16)] = jnp.zeros((16,), _f32)
        return 0
    lax.fori_loop(0, F, _zrow, 0)

    def _zcrow(i, _):
        zcnt[i, pl.ds(0, CW)] = jnp.zeros((CW,), _f32)
        return 0
    lax.fori_loop(0, RPS, _zcrow, 0)

    def _orow(i, _):
        ones[i, pl.ds(0, CW)] = jnp.ones((CW,), _f32)
        return 0
    lax.fori_loop(0, C, _orow, 0)

    for k in range(RPS // F):
        pltpu.sync_copy(zbuf, acc_sh.at[pl.ds(sid * RPS + k * F, F)])
    pltpu.sync_copy(zcnt, cnt_sh.at[pl.ds(sid * RPS, RPS)])
    plsc.subcore_barrier()

    def _scatter(b, j, count_core):
        pltpu.make_async_copy(y_hbm.at[idx_v.at[0, 0]], rows[b],
                              sems[b]).wait()
        pltpu.sync_copy(rows[b], acc_sh.at[idx_v.at[j, 1]], add=True)

        @pl.when(cid == count_core)
        def _():
            pltpu.sync_copy(ones, cnt_sh.at[idx_v.at[j, 1]], add=True)

    for s in range(NSEG):
        if s > 0:
            pltpu.sync_copy(ei_hbm.at[pl.ds(start + s * SEG, SEG)], idx_v)
            for b in range(NB):
                _gissue(b, b)

        def _group(g, _):
            j0 = g * NB
            for b in range(NB):
                _scatter(b, j0 + b, g % 2)
                _gissue(b, j0 + b + NB)
            return 0
        lax.fori_loop(0, SEG // NB - 1, _group, 0)
        for b in range(NB):
            _scatter(b, SEG - NB + b, (SEG // NB - 1) % 2)

    @pl.when(sid < XTRA)
    def _():
        pltpu.sync_copy(ei_hbm.at[pl.ds(start + CPS, 1)], idx_v.at[pl.ds(0, 1)])
        _xform(0)
        pltpu.async_copy(y_hbm.at[idx_v.at[0, 0]], rows[0], sems[0]).wait()
        pltpu.sync_copy(rows[0], acc_sh.at[idx_v.at[0, 1]], add=True)

        @pl.when(cid == 0)
        def _():
            pltpu.sync_copy(ones, cnt_sh.at[idx_v.at[0, 1]], add=True)

    plsc.subcore_barrier()

    @pl.when(cid == 0)
    def _():
        for k in range(RPS // F):
            r0 = sid * RPS + k * F
            pltpu.sync_copy(acc_sh.at[pl.ds(r0, F)],
                            s_out.at[pl.ds(r0, F), pl.ds(0, DH)])
        pltpu.sync_copy(cnt_sh.at[pl.ds(sid * RPS, RPS)],
                        cnt_out.at[pl.ds(sid * RPS, RPS), pl.ds(0, CW)])

    @pl.when(cid == 1)
    def _():
        for k in range(RPS // F):
            r0 = sid * RPS + k * F
            pltpu.sync_copy(acc_sh.at[pl.ds(r0, F)],
                            s_out.at[pl.ds(r0, F), pl.ds(DH, DH)])
        pltpu.sync_copy(cnt_sh.at[pl.ds(sid * RPS, RPS)],
                        cnt_out.at[pl.ds(sid * RPS, RPS), pl.ds(CW, CW)])


_sc_agg = functools.partial(
    pl.kernel,
    out_type=(
        jax.ShapeDtypeStruct((N_PAD, D), _f32),
        jax.ShapeDtypeStruct((N_PAD, D), _f32),
    ),
    mesh=plsc.VectorSubcoreMesh(core_axis_name="c", subcore_axis_name="s",
                                num_cores=NC, num_subcores=NS),
    scratch_types=[
        pltpu.VMEM((SEG, 2, C), jnp.int32),
    ] + [pltpu.VMEM((C, DH), _f32) for _ in range(NB)] + [
        pltpu.VMEM((C, CW), _f32),
        pltpu.VMEM((F, DH), _f32),
        pltpu.VMEM((RPS, CW), _f32),
        pltpu.VMEM_SHARED((N_PAD, DH), _f32),
        pltpu.VMEM_SHARED((N_PAD, CW), _f32),
    ] + [pltpu.SemaphoreType.DMA for _ in range(NB)],
    compiler_params=pltpu.CompilerParams(use_tc_tiling_on_sc=False),
)(_sc_body)


_RB = 1024
_GRID = (N + _RB - 1) // _RB


def _dense0_body(x_ref, wl_ref, wr_ref, bl_ref, br_ref, y_ref, r_ref):
    xb = x_ref[...]
    y_ref[...] = jnp.dot(xb, wl_ref[...], preferred_element_type=_f32)
    r_ref[...] = (jnp.dot(xb, wr_ref[...], preferred_element_type=_f32)
                  + bl_ref[...] + br_ref[...])


_dense0 = pl.pallas_call(
    _dense0_body,
    grid=(_GRID,),
    in_specs=[
        pl.BlockSpec((_RB, D), lambda i: (i, 0)),
        pl.BlockSpec((D, D), lambda i: (0, 0)),
        pl.BlockSpec((D, D), lambda i: (0, 0)),
        pl.BlockSpec((1, D), lambda i: (0, 0)),
        pl.BlockSpec((1, D), lambda i: (0, 0)),
    ],
    out_specs=[
        pl.BlockSpec((_RB, D), lambda i: (i, 0)),
        pl.BlockSpec((_RB, D), lambda i: (i, 0)),
    ],
    out_shape=[
        jax.ShapeDtypeStruct((N, D), _f32),
        jax.ShapeDtypeStruct((N, D), _f32),
    ],
)


def _mean(s_ref, c_ref):
    cnt = c_ref[:, 0:1] + c_ref[:, CW:CW + 1]
    return s_ref[...] / jnp.maximum(cnt, 1.0)


def _combine_mid_body(s_ref, c_ref, r_ref, wl_ref, wr_ref,
                      bl_ref, br_ref, y_ref, rn_ref):
    h = jnp.maximum(_mean(s_ref, c_ref) + r_ref[...], 0.0)
    y_ref[...] = jnp.dot(h, wl_ref[...], preferred_element_type=_f32)
    rn_ref[...] = (jnp.dot(h, wr_ref[...], preferred_element_type=_f32)
                   + bl_ref[...] + br_ref[...])


_combine_mid = pl.pallas_call(
    _combine_mid_body,
    grid=(_GRID,),
    in_specs=[
        pl.BlockSpec((_RB, D), lambda i: (i, 0)),
        pl.BlockSpec((_RB, D), lambda i: (i, 0)),
        pl.BlockSpec((_RB, D), lambda i: (i, 0)),
        pl.BlockSpec((D, D), lambda i: (0, 0)),
        pl.BlockSpec((D, D), lambda i: (0, 0)),
        pl.BlockSpec((1, D), lambda i: (0, 0)),
        pl.BlockSpec((1, D), lambda i: (0, 0)),
    ],
    out_specs=[
        pl.BlockSpec((_RB, D), lambda i: (i, 0)),
        pl.BlockSpec((_RB, D), lambda i: (i, 0)),
    ],
    out_shape=[
        jax.ShapeDtypeStruct((N, D), _f32),
        jax.ShapeDtypeStruct((N, D), _f32),
    ],
)


def _combine_out_body(s_ref, c_ref, r_ref, o_ref):
    z = _mean(s_ref, c_ref) + r_ref[...]
    m = jnp.max(z, axis=-1, keepdims=True)
    e = jnp.exp(z - m)
    o_ref[...] = (z - m) - jnp.log(jnp.sum(e, axis=-1, keepdims=True))


_combine_out = pl.pallas_call(
    _combine_out_body,
    grid=(_GRID,),
    in_specs=[
        pl.BlockSpec((_RB, D), lambda i: (i, 0)),
        pl.BlockSpec((_RB, D), lambda i: (i, 0)),
        pl.BlockSpec((_RB, D), lambda i: (i, 0)),
    ],
    out_specs=pl.BlockSpec((_RB, D), lambda i: (i, 0)),
    out_shape=jax.ShapeDtypeStruct((N, D), _f32),
)


def kernel(x, edge_index_l0, edge_index_l1,
           W_l0, b_l0, W_r0, b_r0,
           W_l1, b_l1, W_r1, b_r1):
    ei0 = edge_index_l0.reshape(2, NCHT, C).transpose(1, 0, 2)
    ei1 = edge_index_l1.reshape(2, NCHT, C).transpose(1, 0, 2)
    bl0 = b_l0.reshape(1, D)
    br0 = b_r0.reshape(1, D)
    bl1 = b_l1.reshape(1, D)
    br1 = b_r1.reshape(1, D)

    y0, r0 = _dense0(x, W_l0, W_r0, bl0, br0)
    s0, c0 = _sc_agg(y0.reshape(2 * N, DH), ei0)
    y1, r1 = _combine_mid(s0, c0, r0, W_l1, W_r1, bl1, br1)
    s1, c1 = _sc_agg(y1.reshape(2 * N, DH), ei1)
    return _combine_out(s1, c1, r1)
